# baseline (speedup 1.0000x reference)
import functools

import jax
import jax.numpy as jnp
from jax import lax
from jax.experimental import pallas as pl
from jax.experimental.pallas import tpu as pltpu
from jax.experimental.pallas import tpu_sc as plsc

_PERM = (
    1, 3, 5, 0, 2, 6, 7, 4,
    10, 8, 12, 13, 15, 14, 11, 9,
    21, 23, 16, 17, 20, 18, 22, 19,
    28, 29, 27, 26, 31, 30, 24, 25,
    38, 37, 33, 35, 34, 39, 36, 32,
    43, 41, 47, 42, 44, 40, 45, 46,
    49, 55, 54, 48, 53, 51, 52, 50,
    61, 56, 58, 63, 57, 59, 60, 62,
)

_B, _C, _T = 128, 64, 4000
_NW = 32
_CH_PW = _C // _NW


def _make_sc_permute():
    mesh = plsc.VectorSubcoreMesh(core_axis_name="c", subcore_axis_name="s")

    @functools.partial(
        pl.kernel,
        mesh=mesh,
        out_type=jax.ShapeDtypeStruct((_C, 1, _T, _B), jnp.float32),
        scratch_types=[
            pltpu.SemaphoreType.DMA,
            pltpu.SemaphoreType.DMA,
        ],
    )
    def sc_permute(in_hbm, out_hbm, s0, s1):
        wid = lax.axis_index("s") * 2 + lax.axis_index("c")
        sems = (s0, s1)
        for w in range(_NW):
            @pl.when(wid == w)
            def _(w=w):
                copies = [
                    pltpu.async_copy(
                        in_hbm.at[pl.ds(_PERM[w * _CH_PW + k], 1)],
                        out_hbm.at[pl.ds(w * _CH_PW + k, 1)],
                        sems[k],
                    )
                    for k in range(_CH_PW)
                ]
                for c in copies:
                    c.wait()

    return sc_permute


def kernel(data_tensor, domain_labels, aux_labels):
    del domain_labels, aux_labels
    x = jnp.transpose(data_tensor, (1, 2, 3, 0))
    y = _make_sc_permute()(x)
    return jnp.transpose(y, (3, 0, 1, 2))


# speedup vs baseline: 35.8454x; 35.8454x over previous
import functools

import jax
import jax.numpy as jnp
from jax import lax
from jax.experimental import pallas as pl
from jax.experimental.pallas import tpu as pltpu
from jax.experimental.pallas import tpu_sc as plsc

_PERM = (
    1, 3, 5, 0, 2, 6, 7, 4,
    10, 8, 12, 13, 15, 14, 11, 9,
    21, 23, 16, 17, 20, 18, 22, 19,
    28, 29, 27, 26, 31, 30, 24, 25,
    38, 37, 33, 35, 34, 39, 36, 32,
    43, 41, 47, 42, 44, 40, 45, 46,
    49, 55, 54, 48, 53, 51, 52, 50,
    61, 56, 58, 63, 57, 59, 60, 62,
)

_B, _C, _T = 128, 64, 4000
_NW = 32
_CH_PW = _C // _NW
_WORDS = tuple(_PERM[2 * w] | (_PERM[2 * w + 1] << 8) for w in range(_NW))

_CHUNK = 400
_NCHUNK = _T // _CHUNK


def _make_sc_permute():
    mesh = plsc.VectorSubcoreMesh(core_axis_name="c", subcore_axis_name="s")

    @functools.partial(
        pl.kernel,
        mesh=mesh,
        out_type=jax.ShapeDtypeStruct((_C, 1, _T, _B), jnp.float32),
        scratch_types=[
            pltpu.VMEM((1, 1, _CHUNK, _B), jnp.float32),
            pltpu.VMEM((1, 1, _CHUNK, _B), jnp.float32),
            pltpu.SemaphoreType.DMA,
            pltpu.SemaphoreType.DMA,
            pltpu.SemaphoreType.DMA,
            pltpu.SemaphoreType.DMA,
        ],
    )
    def sc_permute(in_hbm, out_hbm, buf0, buf1, rs0, rs1, ws0, ws1):
        wid = lax.axis_index("s") * 2 + lax.axis_index("c")
        word = jnp.int32(0)
        for w in range(_NW):
            word = jnp.where(wid == w, jnp.int32(_WORDS[w]), word)
        srcs = (word & 0xFF, word >> 8)
        dsts = (wid * _CH_PW, wid * _CH_PW + 1)
        bufs = (buf0, buf1)
        rsems = (rs0, rs1)
        wsems = (ws0, ws1)

        steps = [(ch, k) for ch in range(_CH_PW) for k in range(_NCHUNK)]

        def read(i):
            ch, k = steps[i]
            p = i % 2
            return pltpu.async_copy(
                in_hbm.at[pl.ds(srcs[ch], 1), :, pl.ds(k * _CHUNK, _CHUNK), :],
                bufs[p],
                rsems[p],
            )

        def write(i):
            ch, k = steps[i]
            p = i % 2
            return pltpu.async_copy(
                bufs[p],
                out_hbm.at[pl.ds(dsts[ch], 1), :, pl.ds(k * _CHUNK, _CHUNK), :],
                wsems[p],
            )

        pending_reads = [read(0), read(1)]
        pending_writes = [None, None]
        for i in range(len(steps)):
            p = i % 2
            pending_reads[p].wait()
            pending_writes[p] = write(i)
            if i + 2 < len(steps):
                pending_writes[p].wait()
                pending_reads[p] = read(i + 2)
        pending_writes[(len(steps) - 1) % 2].wait()

    return sc_permute


def kernel(data_tensor, domain_labels, aux_labels):
    del domain_labels, aux_labels
    x = jnp.transpose(data_tensor, (1, 2, 3, 0))
    y = _make_sc_permute()(x)
    return jnp.transpose(y, (3, 0, 1, 2))


# speedup vs baseline: 36.2530x; 1.0114x over previous
import functools

import jax
import jax.numpy as jnp
from jax import lax
from jax.experimental import pallas as pl
from jax.experimental.pallas import tpu as pltpu
from jax.experimental.pallas import tpu_sc as plsc

_PERM = (
    1, 3, 5, 0, 2, 6, 7, 4,
    10, 8, 12, 13, 15, 14, 11, 9,
    21, 23, 16, 17, 20, 18, 22, 19,
    28, 29, 27, 26, 31, 30, 24, 25,
    38, 37, 33, 35, 34, 39, 36, 32,
    43, 41, 47, 42, 44, 40, 45, 46,
    49, 55, 54, 48, 53, 51, 52, 50,
    61, 56, 58, 63, 57, 59, 60, 62,
)

_B, _C, _T = 128, 64, 4000
_NW = 32
_CH_PW = _C // _NW
_WORDS = tuple(_PERM[2 * w] | (_PERM[2 * w + 1] << 8) for w in range(_NW))

_CHUNK = 200
_NCHUNK = _T // _CHUNK
_NBUF = 4


def _make_sc_permute():
    mesh = plsc.VectorSubcoreMesh(core_axis_name="c", subcore_axis_name="s")

    @functools.partial(
        pl.kernel,
        mesh=mesh,
        out_type=jax.ShapeDtypeStruct((_C, 1, _T, _B), jnp.float32),
        scratch_types=(
            [pltpu.VMEM((1, 1, _CHUNK, _B), jnp.float32)] * _NBUF
            + [pltpu.SemaphoreType.DMA] * (2 * _NBUF)
        ),
    )
    def sc_permute(in_hbm, out_hbm, *scratch):
        bufs = scratch[:_NBUF]
        rsems = scratch[_NBUF:2 * _NBUF]
        wsems = scratch[2 * _NBUF:]
        wid = lax.axis_index("s") * 2 + lax.axis_index("c")
        word = jnp.int32(0)
        for w in range(_NW):
            word = jnp.where(wid == w, jnp.int32(_WORDS[w]), word)
        srcs = (word & 0xFF, word >> 8)
        dsts = (wid * _CH_PW, wid * _CH_PW + 1)

        steps = [(ch, k) for ch in range(_CH_PW) for k in range(_NCHUNK)]
        n = len(steps)

        def read(i):
            ch, k = steps[i]
            p = i % _NBUF
            return pltpu.async_copy(
                in_hbm.at[pl.ds(srcs[ch], 1), :, pl.ds(k * _CHUNK, _CHUNK), :],
                bufs[p],
                rsems[p],
            )

        def write(i):
            ch, k = steps[i]
            p = i % _NBUF
            return pltpu.async_copy(
                bufs[p],
                out_hbm.at[pl.ds(dsts[ch], 1), :, pl.ds(k * _CHUNK, _CHUNK), :],
                wsems[p],
            )

        pending_reads = [None] * _NBUF
        pending_writes = [None] * _NBUF
        lag = 2
        for i in range(n + lag):
            if i < n:
                p = i % _NBUF
                if pending_writes[p] is not None:
                    pending_writes[p].wait()
                pending_reads[p] = read(i)
            if i >= lag:
                j = i - lag
                q = j % _NBUF
                pending_reads[q].wait()
                pending_writes[q] = write(j)
        for j in range(n - _NBUF, n):
            pending_writes[j % _NBUF].wait()

    return sc_permute


def kernel(data_tensor, domain_labels, aux_labels):
    del domain_labels, aux_labels
    x = jnp.transpose(data_tensor, (1, 2, 3, 0))
    y = _make_sc_permute()(x)
    return jnp.transpose(y, (3, 0, 1, 2))


# speedup vs baseline: 37.8413x; 1.0438x over previous
import functools

import jax
import jax.numpy as jnp
from jax import lax
from jax.experimental import pallas as pl
from jax.experimental.pallas import tpu as pltpu
from jax.experimental.pallas import tpu_sc as plsc

_PERM = (
    1, 3, 5, 0, 2, 6, 7, 4,
    10, 8, 12, 13, 15, 14, 11, 9,
    21, 23, 16, 17, 20, 18, 22, 19,
    28, 29, 27, 26, 31, 30, 24, 25,
    38, 37, 33, 35, 34, 39, 36, 32,
    43, 41, 47, 42, 44, 40, 45, 46,
    49, 55, 54, 48, 53, 51, 52, 50,
    61, 56, 58, 63, 57, 59, 60, 62,
)

_B, _C, _T = 128, 64, 4000
_NW = 32
_CH_PW = _C // _NW
_WORDS = tuple(_PERM[2 * w] | (_PERM[2 * w + 1] << 8) for w in range(_NW))

_CHUNK = 200
_NCHUNK = _T // _CHUNK
_NBUF = 4


def _make_sc_permute():
    mesh = plsc.VectorSubcoreMesh(core_axis_name="c", subcore_axis_name="s")

    @functools.partial(
        pl.kernel,
        mesh=mesh,
        out_type=jax.ShapeDtypeStruct((_C, 1, _T, _B), jnp.float32),
        scratch_types=(
            [pltpu.VMEM_SHARED((16, _NBUF, _CHUNK, _B), jnp.float32)]
            + [pltpu.SemaphoreType.DMA] * (2 * _NBUF)
        ),
    )
    def sc_permute(in_hbm, out_hbm, shared, *sems):
        sid = lax.axis_index("s")
        bufs = [
            shared.at[pl.ds(sid, 1), pl.ds(p, 1), :, :] for p in range(_NBUF)
        ]
        rsems = sems[:_NBUF]
        wsems = sems[_NBUF:]
        wid = lax.axis_index("s") * 2 + lax.axis_index("c")
        word = jnp.int32(0)
        for w in range(_NW):
            word = jnp.where(wid == w, jnp.int32(_WORDS[w]), word)
        srcs = (word & 0xFF, word >> 8)
        dsts = (wid * _CH_PW, wid * _CH_PW + 1)

        steps = [(ch, k) for ch in range(_CH_PW) for k in range(_NCHUNK)]
        n = len(steps)

        def read(i):
            ch, k = steps[i]
            p = i % _NBUF
            return pltpu.async_copy(
                in_hbm.at[pl.ds(srcs[ch], 1), :, pl.ds(k * _CHUNK, _CHUNK), :],
                bufs[p],
                rsems[p],
            )

        def write(i):
            ch, k = steps[i]
            p = i % _NBUF
            return pltpu.async_copy(
                bufs[p],
                out_hbm.at[pl.ds(dsts[ch], 1), :, pl.ds(k * _CHUNK, _CHUNK), :],
                wsems[p],
            )

        pending_reads = [None] * _NBUF
        pending_writes = [None] * _NBUF
        lag = 2
        for i in range(n + lag):
            if i < n:
                p = i % _NBUF
                if pending_writes[p] is not None:
                    pending_writes[p].wait()
                pending_reads[p] = read(i)
            if i >= lag:
                j = i - lag
                q = j % _NBUF
                pending_reads[q].wait()
                pending_writes[q] = write(j)
        for j in range(n - _NBUF, n):
            pending_writes[j % _NBUF].wait()

    return sc_permute


def kernel(data_tensor, domain_labels, aux_labels):
    del domain_labels, aux_labels
    x = jnp.transpose(data_tensor, (1, 2, 3, 0))
    y = _make_sc_permute()(x)
    return jnp.transpose(y, (3, 0, 1, 2))
